# Initial kernel scaffold; baseline (speedup 1.0000x reference)
import jax
import jax.numpy as jnp
from jax.experimental import pallas as pl


def kernel(inputs, centroids):
    raise NotImplementedError("write your pallas kernel here")


# speedup vs baseline: 1.1638x; 1.1638x over previous
import functools

import jax
import jax.numpy as jnp
from jax import lax
from jax.experimental import pallas as pl
from jax.experimental.pallas import tpu as pltpu
from jax.experimental.pallas import tpu_sc as plsc

NCENTROIDS = 1024
NCODEBOOKS = 16
SUBVECT = 256
BATCH = 4096

TB = 256
NB = BATCH // TB


def _mm_argmax_kernel(x_ref, cb_ref, out_ref):
    x = x_ref[...]
    cb = cb_ref[0]
    resp = lax.dot_general(
        x, cb, (((1,), (1,)), ((), ())), preferred_element_type=jnp.float32
    )
    mx = jnp.max(resp, axis=1, keepdims=True)
    ii = lax.broadcasted_iota(jnp.int32, resp.shape, 1)
    cand = jnp.where(resp == mx, ii, NCENTROIDS)
    code = jnp.min(cand, axis=1)
    c = pl.program_id(0)
    out_ref[0, 0, :] = code + c * NCENTROIDS


def _compute_codes(inputs, centroids):
    out = pl.pallas_call(
        _mm_argmax_kernel,
        grid=(NCODEBOOKS, NB),
        in_specs=[
            pl.BlockSpec((TB, SUBVECT), lambda c, b: (b, c)),
            pl.BlockSpec((1, NCENTROIDS, SUBVECT), lambda c, b: (c, 0, 0)),
        ],
        out_specs=pl.BlockSpec((1, 1, TB), lambda c, b: (c * NB + b, 0, 0)),
        out_shape=jax.ShapeDtypeStruct((NCODEBOOKS * NB, 1, TB), jnp.int32),
    )(inputs.reshape(BATCH, NCODEBOOKS * SUBVECT), centroids)
    return out.reshape(NCODEBOOKS, BATCH).T


def _make_gather():
    info = plsc.get_sparse_core_info()
    nc, ns = info.num_cores, info.num_subcores
    nw = nc * ns
    b_flat = BATCH * NCODEBOOKS
    b_per_w = b_flat // nw
    ch = 256
    n_chunks = b_per_w // ch
    mesh = plsc.VectorSubcoreMesh(core_axis_name="c", subcore_axis_name="s")

    @functools.partial(
        pl.kernel,
        mesh=mesh,
        out_type=jax.ShapeDtypeStruct((b_flat, SUBVECT), jnp.float32),
        scratch_types=[
            pltpu.VMEM((ch,), jnp.int32),
            pltpu.VMEM((ch, SUBVECT), jnp.float32),
            pltpu.SemaphoreType.DMA,
        ],
    )
    def gather(table_hbm, idx_hbm, out_hbm, idx_v, rows_v, sem):
        wid = lax.axis_index("s") * nc + lax.axis_index("c")
        base = wid * b_per_w
        for j in range(n_chunks):
            off = base + j * ch
            pltpu.sync_copy(idx_hbm.at[pl.ds(off, ch)], idx_v)
            pltpu.async_copy(table_hbm.at[idx_v], rows_v, sem).wait()
            pltpu.sync_copy(rows_v, out_hbm.at[pl.ds(off, ch)])

    return gather


_gather = _make_gather()


def kernel(inputs, centroids):
    neighbour_idxs = _compute_codes(inputs, centroids)
    flat_centroids = centroids.reshape(NCODEBOOKS * NCENTROIDS, SUBVECT)
    flat_idx = neighbour_idxs.reshape(-1)
    rows = _gather(flat_centroids, flat_idx)
    outputs = rows.reshape(BATCH, NCODEBOOKS, SUBVECT)
    return (neighbour_idxs, outputs)


# speedup vs baseline: 2.0630x; 1.7727x over previous
import functools

import jax
import jax.numpy as jnp
from jax import lax
from jax.experimental import pallas as pl
from jax.experimental.pallas import tpu as pltpu
from jax.experimental.pallas import tpu_sc as plsc

NCENTROIDS = 1024
NCODEBOOKS = 16
SUBVECT = 256
BATCH = 4096

TB = 256
NB = BATCH // TB


def _mm_argmax_kernel(x_ref, cb_ref, out_ref):
    cols = []
    for c in range(NCODEBOOKS):
        x = x_ref[:, c * SUBVECT:(c + 1) * SUBVECT]
        resp = lax.dot_general(
            x, cb_ref[c], (((1,), (1,)), ((), ())),
            preferred_element_type=jnp.float32,
        )
        mx = jnp.max(resp, axis=1, keepdims=True)
        ii = lax.broadcasted_iota(jnp.int32, resp.shape, 1)
        cand = jnp.where(resp == mx, ii, NCENTROIDS)
        code = jnp.min(cand, axis=1)
        cols.append((code + c * NCENTROIDS)[:, None])
    out_ref[...] = jnp.concatenate(cols, axis=1)


def _compute_codes(inputs, centroids):
    return pl.pallas_call(
        _mm_argmax_kernel,
        grid=(NB,),
        in_specs=[
            pl.BlockSpec((TB, NCODEBOOKS * SUBVECT), lambda b: (b, 0)),
            pl.BlockSpec(
                (NCODEBOOKS, NCENTROIDS, SUBVECT), lambda b: (0, 0, 0)
            ),
        ],
        out_specs=pl.BlockSpec((TB, NCODEBOOKS), lambda b: (b, 0)),
        out_shape=jax.ShapeDtypeStruct((BATCH, NCODEBOOKS), jnp.int32),
    )(inputs.reshape(BATCH, NCODEBOOKS * SUBVECT), centroids)


def _make_gather():
    info = plsc.get_sparse_core_info()
    nc, ns = info.num_cores, info.num_subcores
    nw = nc * ns
    b_flat = BATCH * NCODEBOOKS
    b_per_w = b_flat // nw
    ch = 128
    n_pairs = b_per_w // (2 * ch)
    mesh = plsc.VectorSubcoreMesh(core_axis_name="c", subcore_axis_name="s")

    @functools.partial(
        pl.kernel,
        mesh=mesh,
        out_type=jax.ShapeDtypeStruct((b_flat, SUBVECT), jnp.float32),
        scratch_types=[
            pltpu.VMEM((b_per_w,), jnp.int32),
            pltpu.VMEM((ch, SUBVECT), jnp.float32),
            pltpu.VMEM((ch, SUBVECT), jnp.float32),
            pltpu.SemaphoreType.DMA,
            pltpu.SemaphoreType.DMA,
        ],
    )
    def gather(table_hbm, idx_hbm, out_hbm, idx_v, rows_a, rows_b, sem_a, sem_b):
        wid = lax.axis_index("s") * nc + lax.axis_index("c")
        base = wid * b_per_w
        pltpu.sync_copy(idx_hbm.at[pl.ds(base, b_per_w)], idx_v)
        pltpu.async_copy(table_hbm.at[idx_v.at[pl.ds(0, ch)]], rows_a, sem_a)

        def body(p, _):
            off = 2 * p * ch
            pltpu.async_copy(
                table_hbm.at[idx_v.at[pl.ds(off + ch, ch)]], rows_b, sem_b
            )
            pltpu.make_async_copy(
                table_hbm.at[idx_v.at[pl.ds(off, ch)]], rows_a, sem_a
            ).wait()
            pltpu.sync_copy(rows_a, out_hbm.at[pl.ds(base + off, ch)])
            nxt = lax.select(p + 1 < n_pairs, off + 2 * ch, 0)
            pltpu.async_copy(
                table_hbm.at[idx_v.at[pl.ds(nxt, ch)]], rows_a, sem_a
            )
            pltpu.make_async_copy(
                table_hbm.at[idx_v.at[pl.ds(off + ch, ch)]], rows_b, sem_b
            ).wait()
            pltpu.sync_copy(rows_b, out_hbm.at[pl.ds(base + off + ch, ch)])
            return 0

        lax.fori_loop(0, n_pairs, body, 0)
        pltpu.make_async_copy(
            table_hbm.at[idx_v.at[pl.ds(0, ch)]], rows_a, sem_a
        ).wait()

    return gather


_gather = _make_gather()


def kernel(inputs, centroids):
    neighbour_idxs = _compute_codes(inputs, centroids)
    flat_centroids = centroids.reshape(NCODEBOOKS * NCENTROIDS, SUBVECT)
    flat_idx = neighbour_idxs.reshape(-1)
    rows = _gather(flat_centroids, flat_idx)
    outputs = rows.reshape(BATCH, NCODEBOOKS, SUBVECT)
    return (neighbour_idxs, outputs)


# speedup vs baseline: 2.0761x; 1.0064x over previous
import functools

import jax
import jax.numpy as jnp
from jax import lax
from jax.experimental import pallas as pl
from jax.experimental.pallas import tpu as pltpu
from jax.experimental.pallas import tpu_sc as plsc

NCENTROIDS = 1024
NCODEBOOKS = 16
SUBVECT = 256
BATCH = 4096

TB = 256
NB = BATCH // TB


def _mm_argmax_kernel(x_ref, cb_ref, out_ref):
    cols = []
    for c in range(NCODEBOOKS):
        x = x_ref[:, c * SUBVECT:(c + 1) * SUBVECT]
        resp = lax.dot_general(
            x, cb_ref[c], (((1,), (1,)), ((), ())),
            preferred_element_type=jnp.float32,
        )
        mx = jnp.max(resp, axis=1, keepdims=True)
        ii = lax.broadcasted_iota(jnp.int32, resp.shape, 1)
        cand = jnp.where(resp == mx, ii, NCENTROIDS)
        code = jnp.min(cand, axis=1)
        cols.append((code + c * NCENTROIDS)[:, None])
    out_ref[...] = jnp.concatenate(cols, axis=1)


def _compute_codes(inputs, centroids):
    return pl.pallas_call(
        _mm_argmax_kernel,
        grid=(NB,),
        in_specs=[
            pl.BlockSpec((TB, NCODEBOOKS * SUBVECT), lambda b: (b, 0)),
            pl.BlockSpec(
                (NCODEBOOKS, NCENTROIDS, SUBVECT), lambda b: (0, 0, 0)
            ),
        ],
        out_specs=pl.BlockSpec((TB, NCODEBOOKS), lambda b: (b, 0)),
        out_shape=jax.ShapeDtypeStruct((BATCH, NCODEBOOKS), jnp.int32),
    )(inputs.reshape(BATCH, NCODEBOOKS * SUBVECT), centroids)


def _make_gather():
    info = plsc.get_sparse_core_info()
    nc, ns = info.num_cores, info.num_subcores
    nw = nc * ns
    rows_per_w = BATCH // nw
    b_per_w = rows_per_w * NCODEBOOKS
    ch = 128
    n_pairs = b_per_w // (2 * ch)
    mesh = plsc.VectorSubcoreMesh(core_axis_name="c", subcore_axis_name="s")

    @functools.partial(
        pl.kernel,
        mesh=mesh,
        out_type=jax.ShapeDtypeStruct(
            (BATCH * NCODEBOOKS, SUBVECT), jnp.float32
        ),
        scratch_types=[
            pltpu.VMEM((rows_per_w, NCODEBOOKS), jnp.int32),
            pltpu.VMEM((b_per_w,), jnp.int32),
            pltpu.VMEM((ch, SUBVECT), jnp.float32),
            pltpu.VMEM((ch, SUBVECT), jnp.float32),
            pltpu.SemaphoreType.DMA,
            pltpu.SemaphoreType.DMA,
        ],
    )
    def gather(table_hbm, idx_hbm, out_hbm, idx2d, idx_v,
               rows_a, rows_b, sem_a, sem_b):
        wid = lax.axis_index("s") * nc + lax.axis_index("c")
        pltpu.sync_copy(idx_hbm.at[pl.ds(wid * rows_per_w, rows_per_w), :], idx2d)

        def fl(r, _):
            idx_v[pl.ds(r * NCODEBOOKS, NCODEBOOKS)] = idx2d[r]
            return 0

        lax.fori_loop(0, rows_per_w, fl, 0)

        base = wid * b_per_w
        pltpu.async_copy(table_hbm.at[idx_v.at[pl.ds(0, ch)]], rows_a, sem_a)

        def body(p, _):
            off = 2 * p * ch
            pltpu.async_copy(
                table_hbm.at[idx_v.at[pl.ds(off + ch, ch)]], rows_b, sem_b
            )
            pltpu.make_async_copy(
                table_hbm.at[idx_v.at[pl.ds(off, ch)]], rows_a, sem_a
            ).wait()
            pltpu.sync_copy(rows_a, out_hbm.at[pl.ds(base + off, ch)])
            nxt = lax.select(p + 1 < n_pairs, off + 2 * ch, 0)
            pltpu.async_copy(
                table_hbm.at[idx_v.at[pl.ds(nxt, ch)]], rows_a, sem_a
            )
            pltpu.make_async_copy(
                table_hbm.at[idx_v.at[pl.ds(off + ch, ch)]], rows_b, sem_b
            ).wait()
            pltpu.sync_copy(rows_b, out_hbm.at[pl.ds(base + off + ch, ch)])
            return 0

        lax.fori_loop(0, n_pairs, body, 0)
        pltpu.make_async_copy(
            table_hbm.at[idx_v.at[pl.ds(0, ch)]], rows_a, sem_a
        ).wait()

    return gather


_gather = _make_gather()


def kernel(inputs, centroids):
    neighbour_idxs = _compute_codes(inputs, centroids)
    flat_centroids = centroids.reshape(NCODEBOOKS * NCENTROIDS, SUBVECT)
    rows = _gather(flat_centroids, neighbour_idxs)
    outputs = rows.reshape(BATCH, NCODEBOOKS, SUBVECT)
    return (neighbour_idxs, outputs)


# speedup vs baseline: 2.4077x; 1.1597x over previous
import functools

import jax
import jax.numpy as jnp
from jax import lax
from jax.experimental import pallas as pl
from jax.experimental.pallas import tpu as pltpu
from jax.experimental.pallas import tpu_sc as plsc

NCENTROIDS = 1024
NCODEBOOKS = 16
SUBVECT = 256
BATCH = 4096

TB = 256
NB = BATCH // TB


def _mm_argmax_kernel(x_ref, cb_ref, out_ref):
    cols = []
    for c in range(NCODEBOOKS):
        x = x_ref[:, c, :]
        resp = lax.dot_general(
            x, cb_ref[c], (((1,), (1,)), ((), ())),
            preferred_element_type=jnp.float32,
        )
        mx = jnp.max(resp, axis=1, keepdims=True)
        ii = lax.broadcasted_iota(jnp.int32, resp.shape, 1)
        cand = jnp.where(resp == mx, ii, NCENTROIDS)
        code = jnp.min(cand, axis=1)
        cols.append((code + c * NCENTROIDS)[:, None])
    out_ref[...] = jnp.concatenate(cols, axis=1)


def _compute_codes(inputs, centroids):
    return pl.pallas_call(
        _mm_argmax_kernel,
        grid=(NB,),
        in_specs=[
            pl.BlockSpec((TB, NCODEBOOKS, SUBVECT), lambda b: (b, 0, 0)),
            pl.BlockSpec(
                (NCODEBOOKS, NCENTROIDS, SUBVECT), lambda b: (0, 0, 0)
            ),
        ],
        out_specs=pl.BlockSpec((TB, NCODEBOOKS), lambda b: (b, 0)),
        out_shape=jax.ShapeDtypeStruct((BATCH, NCODEBOOKS), jnp.int32),
    )(inputs, centroids)


def _make_gather():
    info = plsc.get_sparse_core_info()
    nc, ns = info.num_cores, info.num_subcores
    nw = nc * ns
    rows_per_w = BATCH // nw
    b_per_w = rows_per_w * NCODEBOOKS
    ch = 128
    n_pairs = b_per_w // (2 * ch)
    mesh = plsc.VectorSubcoreMesh(core_axis_name="c", subcore_axis_name="s")

    @functools.partial(
        pl.kernel,
        mesh=mesh,
        out_type=jax.ShapeDtypeStruct(
            (BATCH * NCODEBOOKS, SUBVECT), jnp.float32
        ),
        scratch_types=[
            pltpu.VMEM((rows_per_w, NCODEBOOKS), jnp.int32),
            pltpu.VMEM((b_per_w,), jnp.int32),
            pltpu.VMEM((ch, SUBVECT), jnp.float32),
            pltpu.VMEM((ch, SUBVECT), jnp.float32),
            pltpu.SemaphoreType.DMA,
            pltpu.SemaphoreType.DMA,
        ],
    )
    def gather(table_hbm, idx_hbm, out_hbm, idx2d, idx_v,
               rows_a, rows_b, sem_a, sem_b):
        wid = lax.axis_index("s") * nc + lax.axis_index("c")
        pltpu.sync_copy(idx_hbm.at[pl.ds(wid * rows_per_w, rows_per_w), :], idx2d)

        def fl(r, _):
            idx_v[pl.ds(r * NCODEBOOKS, NCODEBOOKS)] = idx2d[r]
            return 0

        lax.fori_loop(0, rows_per_w, fl, 0)

        base = wid * b_per_w
        pltpu.async_copy(table_hbm.at[idx_v.at[pl.ds(0, ch)]], rows_a, sem_a)

        def body(p, _):
            off = 2 * p * ch
            pltpu.async_copy(
                table_hbm.at[idx_v.at[pl.ds(off + ch, ch)]], rows_b, sem_b
            )
            pltpu.make_async_copy(
                table_hbm.at[idx_v.at[pl.ds(off, ch)]], rows_a, sem_a
            ).wait()
            pltpu.sync_copy(rows_a, out_hbm.at[pl.ds(base + off, ch)])
            nxt = lax.select(p + 1 < n_pairs, off + 2 * ch, 0)
            pltpu.async_copy(
                table_hbm.at[idx_v.at[pl.ds(nxt, ch)]], rows_a, sem_a
            )
            pltpu.make_async_copy(
                table_hbm.at[idx_v.at[pl.ds(off + ch, ch)]], rows_b, sem_b
            ).wait()
            pltpu.sync_copy(rows_b, out_hbm.at[pl.ds(base + off + ch, ch)])
            return 0

        lax.fori_loop(0, n_pairs, body, 0)
        pltpu.make_async_copy(
            table_hbm.at[idx_v.at[pl.ds(0, ch)]], rows_a, sem_a
        ).wait()

    return gather


_gather = _make_gather()


def kernel(inputs, centroids):
    neighbour_idxs = _compute_codes(inputs, centroids)
    flat_centroids = centroids.reshape(NCODEBOOKS * NCENTROIDS, SUBVECT)
    rows = _gather(flat_centroids, neighbour_idxs)
    outputs = rows.reshape(BATCH, NCODEBOOKS, SUBVECT)
    return (neighbour_idxs, outputs)


# speedup vs baseline: 2.7866x; 1.1574x over previous
import functools

import jax
import jax.numpy as jnp
from jax import lax
from jax.experimental import pallas as pl
from jax.experimental.pallas import tpu as pltpu
from jax.experimental.pallas import tpu_sc as plsc

NCENTROIDS = 1024
NCODEBOOKS = 16
SUBVECT = 256
BATCH = 4096

TB = 256
NB = BATCH // TB


def _mm_argmax_kernel(x_ref, cb_ref, out_ref):
    cols = []
    for c in range(NCODEBOOKS):
        resp = lax.dot_general(
            x_ref[:, c, :], cb_ref[c], (((1,), (1,)), ((), ())),
            preferred_element_type=jnp.float32,
        )
        code = jnp.argmax(resp, axis=1).astype(jnp.int32)
        cols.append((code + c * NCENTROIDS)[:, None])
    out_ref[...] = jnp.concatenate(cols, axis=1)


def _compute_codes(inputs, centroids):
    return pl.pallas_call(
        _mm_argmax_kernel,
        grid=(NB,),
        in_specs=[
            pl.BlockSpec((TB, NCODEBOOKS, SUBVECT), lambda b: (b, 0, 0)),
            pl.BlockSpec(
                (NCODEBOOKS, NCENTROIDS, SUBVECT), lambda b: (0, 0, 0)
            ),
        ],
        out_specs=pl.BlockSpec((TB, NCODEBOOKS), lambda b: (b, 0)),
        out_shape=jax.ShapeDtypeStruct((BATCH, NCODEBOOKS), jnp.int32),
    )(inputs, centroids)


def _make_gather():
    info = plsc.get_sparse_core_info()
    nc, ns = info.num_cores, info.num_subcores
    nw = nc * ns
    rows_per_w = BATCH // nw
    b_per_w = rows_per_w * NCODEBOOKS
    ch = 128
    n_pairs = b_per_w // (2 * ch)
    mesh = plsc.VectorSubcoreMesh(core_axis_name="c", subcore_axis_name="s")

    @functools.partial(
        pl.kernel,
        mesh=mesh,
        out_type=jax.ShapeDtypeStruct(
            (BATCH * NCODEBOOKS, SUBVECT), jnp.float32
        ),
        scratch_types=[
            pltpu.VMEM((rows_per_w, NCODEBOOKS), jnp.int32),
            pltpu.VMEM((b_per_w,), jnp.int32),
            pltpu.VMEM((ch, SUBVECT), jnp.float32),
            pltpu.VMEM((ch, SUBVECT), jnp.float32),
            pltpu.SemaphoreType.DMA,
            pltpu.SemaphoreType.DMA,
        ],
    )
    def gather(table_hbm, idx_hbm, out_hbm, idx2d, idx_v,
               rows_a, rows_b, sem_a, sem_b):
        wid = lax.axis_index("s") * nc + lax.axis_index("c")
        pltpu.sync_copy(idx_hbm.at[pl.ds(wid * rows_per_w, rows_per_w), :], idx2d)

        def fl(r, _):
            idx_v[pl.ds(r * NCODEBOOKS, NCODEBOOKS)] = idx2d[r]
            return 0

        lax.fori_loop(0, rows_per_w, fl, 0)

        base = wid * b_per_w
        pltpu.async_copy(table_hbm.at[idx_v.at[pl.ds(0, ch)]], rows_a, sem_a)

        def body(p, _):
            off = 2 * p * ch
            pltpu.async_copy(
                table_hbm.at[idx_v.at[pl.ds(off + ch, ch)]], rows_b, sem_b
            )
            pltpu.make_async_copy(
                table_hbm.at[idx_v.at[pl.ds(off, ch)]], rows_a, sem_a
            ).wait()
            pltpu.sync_copy(rows_a, out_hbm.at[pl.ds(base + off, ch)])
            nxt = lax.select(p + 1 < n_pairs, off + 2 * ch, 0)
            pltpu.async_copy(
                table_hbm.at[idx_v.at[pl.ds(nxt, ch)]], rows_a, sem_a
            )
            pltpu.make_async_copy(
                table_hbm.at[idx_v.at[pl.ds(off + ch, ch)]], rows_b, sem_b
            ).wait()
            pltpu.sync_copy(rows_b, out_hbm.at[pl.ds(base + off + ch, ch)])
            return 0

        lax.fori_loop(0, n_pairs, body, 0)
        pltpu.make_async_copy(
            table_hbm.at[idx_v.at[pl.ds(0, ch)]], rows_a, sem_a
        ).wait()

    return gather


_gather = _make_gather()


def kernel(inputs, centroids):
    neighbour_idxs = _compute_codes(inputs, centroids)
    flat_centroids = centroids.reshape(NCODEBOOKS * NCENTROIDS, SUBVECT)
    rows = _gather(flat_centroids, neighbour_idxs)
    outputs = rows.reshape(BATCH, NCODEBOOKS, SUBVECT)
    return (neighbour_idxs, outputs)


# speedup vs baseline: 2.7914x; 1.0017x over previous
import functools

import jax
import jax.numpy as jnp
from jax import lax
from jax.experimental import pallas as pl
from jax.experimental.pallas import tpu as pltpu
from jax.experimental.pallas import tpu_sc as plsc

NCENTROIDS = 1024
NCODEBOOKS = 16
SUBVECT = 256
BATCH = 4096

TB = 256
NB = BATCH // TB


def _mm_argmax_kernel(x_ref, cb_ref, out_ref):
    cols = []
    for c in range(NCODEBOOKS):
        resp = lax.dot_general(
            x_ref[:, c, :], cb_ref[c], (((1,), (1,)), ((), ())),
            preferred_element_type=jnp.float32,
        )
        code = jnp.argmax(resp, axis=1).astype(jnp.int32)
        cols.append((code + c * NCENTROIDS)[:, None])
    out_ref[...] = jnp.concatenate(cols, axis=1)


def _compute_codes(inputs, centroids):
    return pl.pallas_call(
        _mm_argmax_kernel,
        grid=(NB,),
        in_specs=[
            pl.BlockSpec((TB, NCODEBOOKS, SUBVECT), lambda b: (b, 0, 0)),
            pl.BlockSpec(
                (NCODEBOOKS, NCENTROIDS, SUBVECT), lambda b: (0, 0, 0)
            ),
        ],
        out_specs=pl.BlockSpec((TB, NCODEBOOKS), lambda b: (b, 0)),
        out_shape=jax.ShapeDtypeStruct((BATCH, NCODEBOOKS), jnp.int32),
    )(inputs, centroids)


def _make_gather():
    info = plsc.get_sparse_core_info()
    nc, ns = info.num_cores, info.num_subcores
    nw = nc * ns
    rows_per_w = BATCH // nw
    b_per_w = rows_per_w * NCODEBOOKS
    ch = 64
    n_ch = b_per_w // ch
    nbuf = 4
    mesh = plsc.VectorSubcoreMesh(core_axis_name="c", subcore_axis_name="s")

    @functools.partial(
        pl.kernel,
        mesh=mesh,
        out_type=jax.ShapeDtypeStruct(
            (BATCH * NCODEBOOKS, SUBVECT), jnp.float32
        ),
        scratch_types=[
            pltpu.VMEM((rows_per_w, NCODEBOOKS), jnp.int32),
            pltpu.VMEM((b_per_w,), jnp.int32),
            [pltpu.VMEM((ch, SUBVECT), jnp.float32) for _ in range(nbuf)],
            [pltpu.SemaphoreType.DMA for _ in range(nbuf)],
            [pltpu.SemaphoreType.DMA for _ in range(nbuf)],
        ],
    )
    def gather(table_hbm, idx_hbm, out_hbm, idx2d, idx_v, rows, gsem, wsem):
        wid = lax.axis_index("s") * nc + lax.axis_index("c")
        pltpu.sync_copy(idx_hbm.at[pl.ds(wid * rows_per_w, rows_per_w), :], idx2d)

        def fl(r, _):
            idx_v[pl.ds(r * NCODEBOOKS, NCODEBOOKS)] = idx2d[r]
            return 0

        lax.fori_loop(0, rows_per_w, fl, 0)

        base = wid * b_per_w

        def start_g(j, t):
            pltpu.async_copy(
                table_hbm.at[idx_v.at[pl.ds(j * ch, ch)]], rows[t], gsem[t]
            )

        def wait_g(j, t):
            pltpu.make_async_copy(
                table_hbm.at[idx_v.at[pl.ds(j * ch, ch)]], rows[t], gsem[t]
            ).wait()

        def start_w(j, t):
            pltpu.async_copy(
                rows[t], out_hbm.at[pl.ds(base + j * ch, ch)], wsem[t]
            )

        def wait_w(j, t):
            pltpu.make_async_copy(
                rows[t], out_hbm.at[pl.ds(base + j * ch, ch)], wsem[t]
            ).wait()

        start_g(0, 0)
        start_g(1, 1)
        for j in range(2):
            wait_g(j, j % nbuf)
            start_w(j, j % nbuf)
            start_g(j + 2, (j + 2) % nbuf)
        for j in range(2, 4):
            wait_g(j, j % nbuf)
            start_w(j, j % nbuf)
            wait_w(j - 2, (j - 2) % nbuf)
            start_g(j + 2, (j + 2) % nbuf)

        def body(p, _):
            j0 = 4 * p
            for t in range(4):
                j = j0 + t
                b = (j0 + t) % nbuf
                wait_g(j, t)
                start_w(j, t)
                wait_w(j - 2, (t + 2) % nbuf)
                start_g(j + 2, (t + 2) % nbuf)
            return 0

        lax.fori_loop(1, n_ch // 4 - 1, body, 0)

        for j in range(n_ch - 4, n_ch):
            t = j % nbuf
            wait_g(j, t)
            start_w(j, t)
            wait_w(j - 2, (j - 2) % nbuf)
            if j + 2 < n_ch:
                start_g(j + 2, (j + 2) % nbuf)
        wait_w(n_ch - 2, (n_ch - 2) % nbuf)
        wait_w(n_ch - 1, (n_ch - 1) % nbuf)

    return gather


_gather = _make_gather()


def kernel(inputs, centroids):
    neighbour_idxs = _compute_codes(inputs, centroids)
    flat_centroids = centroids.reshape(NCODEBOOKS * NCENTROIDS, SUBVECT)
    rows = _gather(flat_centroids, neighbour_idxs)
    outputs = rows.reshape(BATCH, NCODEBOOKS, SUBVECT)
    return (neighbour_idxs, outputs)
